# baseline (speedup 1.0000x reference)
import functools

import jax
import jax.numpy as jnp
from jax import lax
from jax.experimental import pallas as pl
from jax.experimental.pallas import tpu as pltpu
from jax.experimental.pallas import tpu_sc as plsc

D_MODEL = 64
SCALE = 8.0

_NC = 2
_NS = 16
_NW = _NC * _NS

_CHUNK = 128


@functools.cache
def _emb_call(b_total, v_half):
    b_per_w = b_total // _NW
    n_chunks = b_per_w // _CHUNK
    mesh = plsc.VectorSubcoreMesh(core_axis_name="c", subcore_axis_name="s")

    @functools.partial(
        pl.kernel,
        mesh=mesh,
        compiler_params=pltpu.CompilerParams(use_tc_tiling_on_sc=True),
        out_type=jax.ShapeDtypeStruct((b_total // 2, 2 * D_MODEL), jnp.float32),
        scratch_types=[
            pltpu.VMEM((b_per_w,), jnp.int32),
            pltpu.VMEM((b_per_w,), jnp.int32),
            pltpu.VMEM((_CHUNK, 2 * D_MODEL), jnp.float32),
            pltpu.VMEM((_CHUNK // 2, 2 * D_MODEL), jnp.float32),
            pltpu.SemaphoreType.DMA,
        ],
    )
    def body(table_hbm, idx_hbm, out_hbm, idx_v, pidx_v, rows_v, out_v, sem):
        wid = lax.axis_index("s") * _NC + lax.axis_index("c")
        base = pl.multiple_of(wid * b_per_w, _CHUNK)
        pltpu.sync_copy(idx_hbm.at[pl.ds(base, b_per_w)], idx_v)

        def pidx_body(i, carry):
            sl = pl.ds(i * 16, 16)
            pidx_v[sl] = lax.shift_right_logical(idx_v[sl], 1)
            return carry

        lax.fori_loop(0, b_per_w // 16, pidx_body, 0)

        def chunk_body(ci, carry):
            off = pl.multiple_of(ci * _CHUNK, _CHUNK)
            pltpu.async_copy(
                table_hbm.at[pidx_v.at[pl.ds(off, _CHUNK)]], rows_v, sem
            ).wait()

            def group_body(g, c2):
                vv = idx_v[pl.ds(off + g * 16, 16)]
                for lane in range(16):
                    v = vv[lane]
                    src = (v & 1) * D_MODEL
                    r = g * 16 + lane
                    q = r // 2
                    dst = (lane % 2) * D_MODEL
                    for c in range(D_MODEL // 16):
                        out_v[q, pl.ds(dst + c * 16, 16)] = (
                            rows_v[r, pl.ds(src + c * 16, 16)] * SCALE
                        )
                return c2

            lax.fori_loop(0, _CHUNK // 16, group_body, 0)
            pltpu.sync_copy(
                out_v,
                out_hbm.at[
                    pl.ds(pl.multiple_of((base + off) // 2, _CHUNK // 2), _CHUNK // 2)
                ],
            )
            return carry

        lax.fori_loop(0, n_chunks, chunk_body, 0)

    return body


def kernel(x, word_emb_weight):
    b_total = x.size
    vocab = word_emb_weight.shape[0]
    idx = x.reshape(b_total)
    table2 = word_emb_weight.reshape(vocab // 2, 2 * D_MODEL)
    out = _emb_call(b_total, vocab // 2)(table2, idx)
    return out.reshape(*x.shape, D_MODEL)


# speedup vs baseline: 1.3909x; 1.3909x over previous
import functools

import jax
import jax.numpy as jnp
from jax import lax
from jax.experimental import pallas as pl
from jax.experimental.pallas import tpu as pltpu
from jax.experimental.pallas import tpu_sc as plsc

D_MODEL = 64
SCALE = 8.0

_NC = 2
_NS = 16
_NW = _NC * _NS

_CHUNK = 128

_TBLK = 2048


def _transpose_block(ta_ref, tb_ref, out_ref):
    out_ref[:, 0:D_MODEL] = jnp.transpose(ta_ref[...], (1, 0))
    out_ref[:, D_MODEL : 2 * D_MODEL] = jnp.transpose(tb_ref[...], (1, 0))


@functools.cache
def _pack_call(vocab):
    grid = (vocab + 2 * _TBLK - 1) // (2 * _TBLK)
    return pl.pallas_call(
        _transpose_block,
        grid=(grid,),
        in_specs=[
            pl.BlockSpec((D_MODEL, _TBLK), lambda i: (0, 2 * i)),
            pl.BlockSpec(
                (D_MODEL, _TBLK),
                lambda i: (0, jnp.minimum(2 * i + 1, 2 * (grid - 1))),
            ),
        ],
        out_specs=pl.BlockSpec((_TBLK, 2 * D_MODEL), lambda i: (i, 0)),
        out_shape=jax.ShapeDtypeStruct((grid * _TBLK, 2 * D_MODEL), jnp.float32),
    )


@functools.cache
def _emb_call(b_total, v_half):
    b_per_w = b_total // _NW
    n_chunks = b_per_w // _CHUNK
    mesh = plsc.VectorSubcoreMesh(core_axis_name="c", subcore_axis_name="s")

    @functools.partial(
        pl.kernel,
        mesh=mesh,
        compiler_params=pltpu.CompilerParams(use_tc_tiling_on_sc=True),
        out_type=jax.ShapeDtypeStruct((b_total // 2, 2 * D_MODEL), jnp.float32),
        scratch_types=[
            pltpu.VMEM((b_per_w,), jnp.int32),
            pltpu.VMEM((b_per_w,), jnp.int32),
            pltpu.VMEM((_CHUNK, 2 * D_MODEL), jnp.float32),
            pltpu.VMEM((_CHUNK // 2, 2 * D_MODEL), jnp.float32),
            pltpu.SemaphoreType.DMA,
        ],
    )
    def body(table_hbm, idx_hbm, out_hbm, idx_v, pidx_v, rows0, out_v, sem0):
        wid = lax.axis_index("s") * _NC + lax.axis_index("c")
        base = pl.multiple_of(wid * b_per_w, _CHUNK)
        pltpu.sync_copy(idx_hbm.at[pl.ds(base, b_per_w)], idx_v)

        def pidx_body(i, carry):
            sl = pl.ds(i * 16, 16)
            v = idx_v[sl]
            pidx_v[sl] = lax.shift_left(
                lax.shift_right_logical(v, 12), 11
            ) | (v & 2047)
            return carry

        lax.fori_loop(0, b_per_w // 16, pidx_body, 0)

        def chunk_body(ci, carry):
            off = pl.multiple_of(ci * _CHUNK, _CHUNK)
            rows_v = rows0
            pltpu.async_copy(
                table_hbm.at[pidx_v.at[pl.ds(off, _CHUNK)]], rows_v, sem0
            ).wait()

            def group_body(g, c2):
                vv = idx_v[pl.ds(off + g * 16, 16)]
                for lane in range(16):
                    v = vv[lane]
                    src = lax.shift_right_logical(v, 5) & D_MODEL
                    r = g * 16 + lane
                    q = r // 2
                    dst = (lane % 2) * D_MODEL
                    for c in range(D_MODEL // 16):
                        out_v[q, pl.ds(dst + c * 16, 16)] = (
                            rows_v[r, pl.ds(src + c * 16, 16)] * SCALE
                        )
                return c2

            lax.fori_loop(0, _CHUNK // 16, group_body, 0)
            pltpu.sync_copy(
                out_v,
                out_hbm.at[
                    pl.ds(pl.multiple_of((base + off) // 2, _CHUNK // 2),
                          _CHUNK // 2)
                ],
            )
            return carry

        lax.fori_loop(0, n_chunks, chunk_body, 0)

    return body


def kernel(x, word_emb_weight):
    b_total = x.size
    vocab = word_emb_weight.shape[0]
    t_t = word_emb_weight.T
    table2 = _pack_call(vocab)(t_t, t_t)
    idx = x.reshape(b_total)
    out = _emb_call(b_total, vocab // 2)(table2, idx)
    return out.reshape(*x.shape, D_MODEL)


# speedup vs baseline: 1.8263x; 1.3130x over previous
import functools

import jax
import jax.numpy as jnp
from jax import lax
from jax.experimental import pallas as pl
from jax.experimental.pallas import tpu as pltpu
from jax.experimental.pallas import tpu_sc as plsc

D_MODEL = 64
SCALE = 8.0

_NC = 2
_NS = 16
_NW = _NC * _NS

_CHUNK = 128

_TBLK = 4096
_TSH = 13


def _transpose_block(ta_ref, tb_ref, out_ref):
    out_ref[:, 0:D_MODEL] = jnp.transpose(ta_ref[...], (1, 0)) * SCALE
    out_ref[:, D_MODEL : 2 * D_MODEL] = (
        jnp.transpose(tb_ref[...], (1, 0)) * SCALE
    )


@functools.cache
def _pack_call(vocab):
    grid = (vocab + 2 * _TBLK - 1) // (2 * _TBLK)
    return pl.pallas_call(
        _transpose_block,
        grid=(grid,),
        in_specs=[
            pl.BlockSpec((D_MODEL, _TBLK), lambda i: (0, 2 * i)),
            pl.BlockSpec(
                (D_MODEL, _TBLK),
                lambda i: (0, jnp.minimum(2 * i + 1, 2 * (grid - 1))),
            ),
        ],
        out_specs=pl.BlockSpec((_TBLK, 2 * D_MODEL), lambda i: (i, 0)),
        out_shape=jax.ShapeDtypeStruct((grid * _TBLK, 2 * D_MODEL), jnp.float32),
    )


@functools.cache
def _emb_call(b_total, v_half):
    b_per_w = b_total // _NW
    n_chunks = b_per_w // _CHUNK
    mesh = plsc.VectorSubcoreMesh(core_axis_name="c", subcore_axis_name="s")

    @functools.partial(
        pl.kernel,
        mesh=mesh,
        compiler_params=pltpu.CompilerParams(use_tc_tiling_on_sc=True),
        out_type=jax.ShapeDtypeStruct((b_total // 2, 2 * D_MODEL), jnp.float32),
        scratch_types=[
            pltpu.VMEM((b_per_w,), jnp.int32),
            pltpu.VMEM((b_per_w,), jnp.int32),
            pltpu.VMEM((_CHUNK, 2 * D_MODEL), jnp.float32),
            pltpu.VMEM((_CHUNK, 2 * D_MODEL), jnp.float32),
            pltpu.VMEM((_CHUNK // 2, 2 * D_MODEL), jnp.float32),
            pltpu.SemaphoreType.DMA,
            pltpu.SemaphoreType.DMA,
        ],
    )
    def body(table_hbm, idx_hbm, out_hbm, idx_v, pidx_v, rows0, rows1, out_v,
             sem0, sem1):
        wid = lax.axis_index("s") * _NC + lax.axis_index("c")
        base = pl.multiple_of(wid * b_per_w, _CHUNK)
        pltpu.sync_copy(idx_hbm.at[pl.ds(base, b_per_w)], idx_v)

        def pidx_body(i, carry):
            sl = pl.ds(i * 16, 16)
            v = idx_v[sl]
            pidx_v[sl] = lax.shift_left(
                lax.shift_right_logical(v, _TSH), _TSH - 1
            ) | (v & (_TBLK - 1))
            return carry

        lax.fori_loop(0, b_per_w // 16, pidx_body, 0)

        bufs = (rows0, rows1)
        sems = (sem0, sem1)

        def start(ci, b):
            off = pl.multiple_of(ci * _CHUNK, _CHUNK)
            pltpu.async_copy(
                table_hbm.at[pidx_v.at[pl.ds(off, _CHUNK)]], bufs[b], sems[b]
            )

        def wait(ci, b):
            off = pl.multiple_of(ci * _CHUNK, _CHUNK)
            pltpu.make_async_copy(
                table_hbm.at[pidx_v.at[pl.ds(off, _CHUNK)]], bufs[b], sems[b]
            ).wait()

        def work(ci, b):
            off = pl.multiple_of(ci * _CHUNK, _CHUNK)
            rows_v = bufs[b]

            def group_body(g, c2):
                vv = idx_v[pl.ds(off + g * 16, 16)]
                for lane in range(16):
                    v = vv[lane]
                    src = lax.shift_right_logical(v, _TSH - 7) & D_MODEL
                    r = g * 16 + lane
                    q = r // 2
                    dst = (lane % 2) * D_MODEL
                    for c in range(D_MODEL // 16):
                        out_v[q, pl.ds(dst + c * 16, 16)] = rows_v[
                            r, pl.ds(src + c * 16, 16)
                        ]
                return c2

            lax.fori_loop(0, _CHUNK // 16, group_body, 0)
            pltpu.sync_copy(
                out_v,
                out_hbm.at[
                    pl.ds(pl.multiple_of((base + off) // 2, _CHUNK // 2),
                          _CHUNK // 2)
                ],
            )

        start(0, 0)

        def pair_body(cio, carry):
            ci0 = cio * 2
            start(ci0 + 1, 1)
            wait(ci0, 0)
            work(ci0, 0)
            start(ci0 + 2, 0)
            wait(ci0 + 1, 1)
            work(ci0 + 1, 1)
            return carry

        lax.fori_loop(0, n_chunks // 2 - 1, pair_body, 0)

        last = n_chunks - 2
        start(last + 1, 1)
        wait(last, 0)
        work(last, 0)
        wait(last + 1, 1)
        work(last + 1, 1)

    return body


def kernel(x, word_emb_weight):
    b_total = x.size
    vocab = word_emb_weight.shape[0]
    t_t = word_emb_weight.T
    table2 = _pack_call(vocab)(t_t, t_t)
    idx = x.reshape(b_total)
    out = _emb_call(b_total, vocab // 2)(table2, idx)
    return out.reshape(*x.shape, D_MODEL)


# speedup vs baseline: 1.9525x; 1.0691x over previous
import functools

import jax
import jax.numpy as jnp
from jax import lax
from jax.experimental import pallas as pl
from jax.experimental.pallas import tpu as pltpu
from jax.experimental.pallas import tpu_sc as plsc

D_MODEL = 64
SCALE = 8.0

_NC = 2
_NS = 16
_NW = _NC * _NS

_CHUNK = 128

_TBLK = 8192
_TSH = 14


def _transpose_block(ta_ref, tb_ref, out_ref):
    out_ref[:, 0:D_MODEL] = jnp.transpose(ta_ref[...], (1, 0)) * SCALE
    out_ref[:, D_MODEL : 2 * D_MODEL] = (
        jnp.transpose(tb_ref[...], (1, 0)) * SCALE
    )


@functools.cache
def _pack_call(vocab):
    grid = (vocab + 2 * _TBLK - 1) // (2 * _TBLK)
    return pl.pallas_call(
        _transpose_block,
        grid=(grid,),
        in_specs=[
            pl.BlockSpec((D_MODEL, _TBLK), lambda i: (0, 2 * i)),
            pl.BlockSpec(
                (D_MODEL, _TBLK),
                lambda i: (0, jnp.minimum(2 * i + 1, 2 * (grid - 1))),
            ),
        ],
        out_specs=pl.BlockSpec((_TBLK, 2 * D_MODEL), lambda i: (i, 0)),
        out_shape=jax.ShapeDtypeStruct((grid * _TBLK, 2 * D_MODEL), jnp.float32),
    )


@functools.cache
def _emb_call(b_total, v_half):
    b_per_w = b_total // _NW
    n_chunks = b_per_w // _CHUNK
    mesh = plsc.VectorSubcoreMesh(core_axis_name="c", subcore_axis_name="s")

    @functools.partial(
        pl.kernel,
        mesh=mesh,
        compiler_params=pltpu.CompilerParams(use_tc_tiling_on_sc=True),
        out_type=jax.ShapeDtypeStruct((b_total // 2, 2 * D_MODEL), jnp.float32),
        scratch_types=[
            pltpu.VMEM((b_per_w,), jnp.int32),
            pltpu.VMEM((b_per_w,), jnp.int32),
            pltpu.VMEM((_CHUNK, 2 * D_MODEL), jnp.float32),
            pltpu.VMEM((_CHUNK, 2 * D_MODEL), jnp.float32),
            pltpu.VMEM((_CHUNK // 2, 2 * D_MODEL), jnp.float32),
            pltpu.SemaphoreType.DMA,
            pltpu.SemaphoreType.DMA,
        ],
    )
    def body(table_hbm, idx_hbm, out_hbm, idx_v, pidx_v, rows0, rows1, out_v,
             sem0, sem1):
        wid = lax.axis_index("s") * _NC + lax.axis_index("c")
        base = pl.multiple_of(wid * b_per_w, _CHUNK)
        pltpu.sync_copy(idx_hbm.at[pl.ds(base, b_per_w)], idx_v)

        def pidx_body(i, carry):
            sl = pl.ds(i * 16, 16)
            v = idx_v[sl]
            pidx_v[sl] = lax.shift_left(
                lax.shift_right_logical(v, _TSH), _TSH - 1
            ) | (v & (_TBLK - 1))
            return carry

        lax.fori_loop(0, b_per_w // 16, pidx_body, 0)

        bufs = (rows0, rows1)
        sems = (sem0, sem1)

        def start(ci, b):
            off = pl.multiple_of(ci * _CHUNK, _CHUNK)
            pltpu.async_copy(
                table_hbm.at[pidx_v.at[pl.ds(off, _CHUNK)]], bufs[b], sems[b]
            )

        def wait(ci, b):
            off = pl.multiple_of(ci * _CHUNK, _CHUNK)
            pltpu.make_async_copy(
                table_hbm.at[pidx_v.at[pl.ds(off, _CHUNK)]], bufs[b], sems[b]
            ).wait()

        def work(ci, b):
            off = pl.multiple_of(ci * _CHUNK, _CHUNK)
            rows_v = bufs[b]

            def group_body(g, c2):
                vv = idx_v[pl.ds(off + g * 16, 16)]
                for lane in range(16):
                    v = vv[lane]
                    src = lax.shift_right_logical(v, _TSH - 7) & D_MODEL
                    r = g * 16 + lane
                    q = r // 2
                    dst = (lane % 2) * D_MODEL
                    for c in range(D_MODEL // 16):
                        out_v[q, pl.ds(dst + c * 16, 16)] = rows_v[
                            r, pl.ds(src + c * 16, 16)
                        ]
                return c2

            lax.fori_loop(0, _CHUNK // 16, group_body, 0)
            pltpu.sync_copy(
                out_v,
                out_hbm.at[
                    pl.ds(pl.multiple_of((base + off) // 2, _CHUNK // 2),
                          _CHUNK // 2)
                ],
            )

        start(0, 0)

        def pair_body(cio, carry):
            ci0 = cio * 2
            start(ci0 + 1, 1)
            wait(ci0, 0)
            work(ci0, 0)
            start(ci0 + 2, 0)
            wait(ci0 + 1, 1)
            work(ci0 + 1, 1)
            return carry

        lax.fori_loop(0, n_chunks // 2 - 1, pair_body, 0)

        last = n_chunks - 2
        start(last + 1, 1)
        wait(last, 0)
        work(last, 0)
        wait(last + 1, 1)
        work(last + 1, 1)

    return body


def kernel(x, word_emb_weight):
    b_total = x.size
    vocab = word_emb_weight.shape[0]
    t_t = word_emb_weight.T
    table2 = _pack_call(vocab)(t_t, t_t)
    idx = x.reshape(b_total)
    out = _emb_call(b_total, vocab // 2)(table2, idx)
    return out.reshape(*x.shape, D_MODEL)
